# Initial kernel scaffold; baseline (speedup 1.0000x reference)
import jax
import jax.numpy as jnp
from jax.experimental import pallas as pl


def kernel(x, edge_index, enc_w1, enc_b1, enc_w2, enc_b2, gcn_w0, gcn_b0, gcn_w1, gcn_b1, gcn_w2, gcn_b2, cl_w1, cl_b1, cl_w2, cl_b2, so_w1, so_b1, so_w2, so_b2):
    raise NotImplementedError("write your pallas kernel here")


# speedup vs baseline: 11.8191x; 11.8191x over previous
import functools

import jax
import jax.numpy as jnp
from jax import lax
from jax.experimental import pallas as pl
from jax.experimental.pallas import tpu as pltpu
from jax.experimental.pallas import tpu_sc as plsc

N_NODES = 10000
N_PAD = 10240
N_EDGES = 320000
HID = 128
N_SC = 2
N_SUB = 16
N_WORK = N_SC * N_SUB
CHUNK = 80
EDGES_PER_WORKER = N_EDGES // N_WORK
CHUNKS_PER_WORKER = EDGES_PER_WORKER // CHUNK
ROWS_PER_SUB = N_PAD // N_SUB
ZROWS = 128


def _fill_zeros(buf, rows, cols):
    z = jnp.zeros((16,), jnp.float32)

    def body(i, _):
        r = i // (cols // 16)
        c = (i % (cols // 16)) * 16
        buf[r, pl.ds(c, 16)] = z
        return 0

    lax.fori_loop(0, rows * (cols // 16), body, 0)


def _sc_mesh():
    return plsc.VectorSubcoreMesh(
        core_axis_name="c", subcore_axis_name="s",
        num_cores=N_SC, num_subcores=N_SUB)


HHID = HID // 2
AGG_CHUNKS = N_EDGES // N_SUB // CHUNK


@functools.lru_cache(maxsize=None)
def _agg_kernel():
    @functools.partial(
        pl.kernel,
        mesh=_sc_mesh(),
        compiler_params=pltpu.CompilerParams(use_tc_tiling_on_sc=False),
        out_type=jax.ShapeDtypeStruct((N_SC, N_PAD, HHID), jnp.float32),
        scratch_types=[
            pltpu.VMEM((AGG_CHUNKS, CHUNK), jnp.int32),
            pltpu.VMEM((AGG_CHUNKS, CHUNK), jnp.int32),
            pltpu.VMEM((CHUNK, HHID), jnp.float32),
            pltpu.VMEM((ZROWS, HHID), jnp.float32),
            pltpu.VMEM_SHARED((N_PAD, HHID), jnp.float32),
            pltpu.SemaphoreType.DMA,
        ],
    )
    def agg(g2, src3d, dst3d, out, src_v, dst_v, rows_v, zbuf, acc, sem):
        c = lax.axis_index("c")
        s = lax.axis_index("s")
        _fill_zeros(zbuf, ZROWS, HHID)
        for k in range(ROWS_PER_SUB // ZROWS):
            pltpu.sync_copy(zbuf, acc.at[pl.ds(s * ROWS_PER_SUB + k * ZROWS, ZROWS)])
        plsc.subcore_barrier()
        pltpu.sync_copy(src3d.at[s], src_v)
        pltpu.sync_copy(dst3d.at[s], dst_v)
        g_half = g2.at[c]

        def chunk(j, _):
            pltpu.async_copy(g_half.at[src_v.at[j]], rows_v, sem).wait()
            pltpu.sync_copy(rows_v, acc.at[dst_v.at[j]], add=True)
            return 0

        lax.fori_loop(0, AGG_CHUNKS, chunk, 0)
        plsc.subcore_barrier()
        pltpu.sync_copy(acc.at[pl.ds(s * ROWS_PER_SUB, ROWS_PER_SUB)],
                        out.at[c, pl.ds(s * ROWS_PER_SUB, ROWS_PER_SUB)])

    return agg


@functools.lru_cache(maxsize=None)
def _deg_kernel():
    @functools.partial(
        pl.kernel,
        mesh=_sc_mesh(),
        compiler_params=pltpu.CompilerParams(use_tc_tiling_on_sc=False),
        out_type=jax.ShapeDtypeStruct((N_SC, N_PAD, 16), jnp.float32),
        scratch_types=[
            pltpu.VMEM((CHUNKS_PER_WORKER, CHUNK), jnp.int32),
            pltpu.VMEM((CHUNK, 16), jnp.float32),
            pltpu.VMEM((ROWS_PER_SUB, 16), jnp.float32),
            pltpu.VMEM_SHARED((N_PAD, 16), jnp.float32),
        ],
    )
    def deg(dst3d, out, dst_v, ones_v, zbuf, acc):
        c = lax.axis_index("c")
        s = lax.axis_index("s")
        wid = s * N_SC + c
        _fill_zeros(zbuf, ROWS_PER_SUB, 16)
        pltpu.sync_copy(zbuf, acc.at[pl.ds(s * ROWS_PER_SUB, ROWS_PER_SUB)])
        e0 = jnp.where(lax.iota(jnp.int32, 16) == 0, 1.0, 0.0).astype(jnp.float32)

        def fill_ones(i, _):
            ones_v[i, pl.ds(0, 16)] = e0
            return 0

        lax.fori_loop(0, CHUNK, fill_ones, 0)
        plsc.subcore_barrier()
        pltpu.sync_copy(dst3d.at[wid], dst_v)

        def chunk(j, _):
            pltpu.sync_copy(ones_v, acc.at[dst_v.at[j]], add=True)
            return 0

        lax.fori_loop(0, CHUNKS_PER_WORKER, chunk, 0)
        plsc.subcore_barrier()
        pltpu.sync_copy(acc.at[pl.ds(s * ROWS_PER_SUB, ROWS_PER_SUB)],
                        out.at[c, pl.ds(s * ROWS_PER_SUB, ROWS_PER_SUB)])

    return deg


def kernel(x, edge_index, enc_w1, enc_b1, enc_w2, enc_b2,
           gcn_w0, gcn_b0, gcn_w1, gcn_b1, gcn_w2, gcn_b2,
           cl_w1, cl_b1, cl_w2, cl_b2, so_w1, so_b1, so_w2, so_b2):
    src = edge_index[0]
    dst = edge_index[1]
    src3d = src.reshape(N_WORK, CHUNKS_PER_WORKER, CHUNK)
    dst3d = dst.reshape(N_WORK, CHUNKS_PER_WORKER, CHUNK)
    srcA = src.reshape(N_SUB, AGG_CHUNKS, CHUNK)
    dstA = dst.reshape(N_SUB, AGG_CHUNKS, CHUNK)

    deg_parts = _deg_kernel()(dst3d)
    deg = deg_parts[0, :N_NODES, 0] + deg_parts[1, :N_NODES, 0] + 1.0
    dis = lax.rsqrt(deg)

    h = jax.nn.relu(x @ enc_w1 + enc_b1) @ enc_w2 + enc_b2

    gws = [(gcn_w0, gcn_b0), (gcn_w1, gcn_b1), (gcn_w2, gcn_b2)]
    for i, (W, b) in enumerate(gws):
        g = (h @ W) * dis[:, None]
        g2 = jnp.stack([g[:, :HHID], g[:, HHID:]])
        parts = _agg_kernel()(g2, srcA, dstA)
        asum = jnp.concatenate([parts[0, :N_NODES], parts[1, :N_NODES]], axis=-1)
        h_new = jax.nn.relu(dis[:, None] * (asum + g) + b)
        h = h + h_new if i > 0 else h_new

    cluster_logits = jax.nn.relu(h @ cl_w1 + cl_b1) @ cl_w2 + cl_b2
    cluster_probs = jax.nn.softmax(cluster_logits, axis=-1)
    cluster_features = jnp.broadcast_to(h.mean(axis=0, keepdims=True), h.shape)
    combined = jnp.concatenate([h, cluster_features], axis=-1)
    solar_scores = jax.nn.sigmoid(
        jax.nn.relu(combined @ so_w1 + so_b1) @ so_w2 + so_b2)[:, 0]
    return (cluster_logits, cluster_probs, solar_scores, h)
